# baseline (device time: 4576797 ns/iter reference)
import jax
import jax.numpy as jnp
from jax import lax
from jax.experimental import pallas as pl
from jax.experimental.pallas import tpu as pltpu

N = 4
M = 4096
D = 4096
CR = 256
NC = M // CR
TR = 128
EPS = 1e-6
F32 = jnp.float32


def kernel(partial, resid, gamma):
    gamma2 = gamma.reshape(1, D)

    def body(x_hbm, resid_hbm, gamma_ref, out_hbm,
             xbuf, residbuf, rrecv, lrecv, rsend, lsend, obuf,
             xsems, ressems, osems,
             rsend_sems, rrecv_sems, lsend_sems, lrecv_sems,
             rcred, lcred):
        xi = lax.axis_index("x")
        yi = lax.axis_index("y")
        zi = lax.axis_index("z")
        right = (xi, yi, jnp.minimum(zi + 1, N - 1))
        left = (xi, yi, jnp.maximum(zi - 1, 0))
        is_z0 = zi == 0
        is_z3 = zi == N - 1
        not_z0 = jnp.logical_not(is_z0)
        not_z3 = jnp.logical_not(is_z3)
        is_mid = jnp.logical_and(not_z0, not_z3)
        is_edge = jnp.logical_or(is_z0, is_z3)

        bar = pltpu.get_barrier_semaphore()

        @pl.when(not_z0)
        def _():
            pl.semaphore_signal(bar, 1, device_id=left)

        @pl.when(not_z3)
        def _():
            pl.semaphore_signal(bar, 1, device_id=right)

        @pl.when(is_edge)
        def _():
            pl.semaphore_wait(bar, 1)

        @pl.when(is_mid)
        def _():
            pl.semaphore_wait(bar, 2)

        def desc_r(src, s):
            return pltpu.make_async_remote_copy(
                src_ref=src, dst_ref=rrecv.at[s],
                send_sem=rsend_sems.at[s], recv_sem=rrecv_sems.at[s],
                device_id=right)

        def desc_l(src, s):
            return pltpu.make_async_remote_copy(
                src_ref=src, dst_ref=lrecv.at[s],
                send_sem=lsend_sems.at[s], recv_sem=lrecv_sems.at[s],
                device_id=left)

        def copy(src, dst, sem):
            return pltpu.make_async_copy(src, dst, sem)

        def rows(c):
            return pl.ds(c * CR, CR)

        def xload(c, s):
            return copy(x_hbm.at[0, rows(c), :], xbuf.at[s], xsems.at[s])

        def resload(c, s):
            return copy(resid_hbm.at[rows(c), :], residbuf.at[s],
                        ressems.at[s])

        def ostore(c, s):
            return copy(obuf.at[s], out_hbm.at[rows(c), :], osems.at[s])

        for c in range(2):
            xload(c, c).start()
            resload(c, c).start()

        def chunk_step(c, _):
            s = lax.rem(c, 2)
            warm = c >= 2
            more = c + 2 < NC
            cp2 = jnp.minimum(c + 2, NC - 1)
            cm2 = jnp.maximum(c - 2, 0)

            xload(c, s).wait()

            @pl.when(jnp.logical_and(is_z0, warm))
            def _():
                pl.semaphore_wait(rcred, 1)

            @pl.when(is_z0)
            def _():
                desc_r(xbuf.at[s], s).start()

            @pl.when(not_z0)
            def _():
                desc_r(rsend.at[s], s).wait_recv()

            @pl.when(jnp.logical_and(is_mid, warm))
            def _():
                desc_r(rsend.at[s], s).wait_send()

            @pl.when(is_mid)
            def _():
                for t in range(CR // TR):
                    ts = pl.ds(t * TR, TR)
                    rsend[s, ts, :] = rrecv[s, ts, :] + xbuf[s, ts, :]
                pl.semaphore_signal(rcred, 1, device_id=left)

            @pl.when(jnp.logical_and(is_mid, warm))
            def _():
                pl.semaphore_wait(rcred, 1)

            @pl.when(is_mid)
            def _():
                desc_r(rsend.at[s], s).start()

            @pl.when(jnp.logical_and(is_z3, warm))
            def _():
                pl.semaphore_wait(lcred, 1)

            @pl.when(is_z3)
            def _():
                desc_l(xbuf.at[s], s).start()

            @pl.when(not_z3)
            def _():
                desc_l(lsend.at[s], s).wait_recv()

            @pl.when(jnp.logical_and(is_mid, warm))
            def _():
                desc_l(lsend.at[s], s).wait_send()

            @pl.when(is_mid)
            def _():
                for t in range(CR // TR):
                    ts = pl.ds(t * TR, TR)
                    lsend[s, ts, :] = lrecv[s, ts, :] + xbuf[s, ts, :]

            @pl.when(jnp.logical_and(is_mid, warm))
            def _():
                pl.semaphore_wait(lcred, 1)

            @pl.when(is_mid)
            def _():
                desc_l(lsend.at[s], s).start()

            resload(c, s).wait()

            @pl.when(warm)
            def _():
                ostore(cm2, s).wait()

            def norm(y_of_ts):
                for t in range(CR // TR):
                    ts = pl.ds(t * TR, TR)
                    y = y_of_ts(ts) + residbuf[s, ts, :]
                    ms = jnp.mean(y * y, axis=-1, keepdims=True)
                    obuf[s, ts, :] = (
                        (y * lax.rsqrt(ms + EPS)) * gamma_ref[...])

            @pl.when(is_z0)
            def _():
                norm(lambda ts: xbuf[s, ts, :] + lrecv[s, ts, :])

            @pl.when(is_mid)
            def _():
                norm(lambda ts: rsend[s, ts, :] + lrecv[s, ts, :])

            @pl.when(is_z3)
            def _():
                norm(lambda ts: xbuf[s, ts, :] + rrecv[s, ts, :])

            @pl.when(not_z3)
            def _():
                pl.semaphore_signal(lcred, 1, device_id=right)

            @pl.when(is_z3)
            def _():
                pl.semaphore_signal(rcred, 1, device_id=left)

            ostore(c, s).start()

            @pl.when(jnp.logical_and(more, is_z0))
            def _():
                desc_r(xbuf.at[s], s).wait_send()

            @pl.when(jnp.logical_and(more, is_z3))
            def _():
                desc_l(xbuf.at[s], s).wait_send()

            @pl.when(more)
            def _():
                xload(cp2, s).start()
                resload(cp2, s).start()

            return 0

        lax.fori_loop(0, NC, chunk_step, 0)

        for c in (NC - 2, NC - 1):
            ostore(c, c % 2).wait()

        @pl.when(is_z0)
        def _():
            for s in range(2):
                desc_r(xbuf.at[s], s).wait_send()
            pl.semaphore_wait(rcred, 2)

        @pl.when(is_z3)
        def _():
            for s in range(2):
                desc_l(xbuf.at[s], s).wait_send()
            pl.semaphore_wait(lcred, 2)

        @pl.when(is_mid)
        def _():
            for s in range(2):
                desc_r(rsend.at[s], s).wait_send()
                desc_l(lsend.at[s], s).wait_send()
            pl.semaphore_wait(rcred, 2)
            pl.semaphore_wait(lcred, 2)

    return pl.pallas_call(
        body,
        out_shape=jax.ShapeDtypeStruct((M, D), F32),
        in_specs=[
            pl.BlockSpec(memory_space=pltpu.MemorySpace.HBM),
            pl.BlockSpec(memory_space=pltpu.MemorySpace.HBM),
            pl.BlockSpec(memory_space=pltpu.MemorySpace.VMEM),
        ],
        out_specs=pl.BlockSpec(memory_space=pltpu.MemorySpace.HBM),
        scratch_shapes=[
            pltpu.VMEM((2, CR, D), F32),
            pltpu.VMEM((2, CR, D), F32),
            pltpu.VMEM((2, CR, D), F32),
            pltpu.VMEM((2, CR, D), F32),
            pltpu.VMEM((2, CR, D), F32),
            pltpu.VMEM((2, CR, D), F32),
            pltpu.VMEM((2, CR, D), F32),
            pltpu.SemaphoreType.DMA((2,)),
            pltpu.SemaphoreType.DMA((2,)),
            pltpu.SemaphoreType.DMA((2,)),
            pltpu.SemaphoreType.DMA((2,)),
            pltpu.SemaphoreType.DMA((2,)),
            pltpu.SemaphoreType.DMA((2,)),
            pltpu.SemaphoreType.DMA((2,)),
            pltpu.SemaphoreType.REGULAR,
            pltpu.SemaphoreType.REGULAR,
        ],
        compiler_params=pltpu.CompilerParams(
            collective_id=0, vmem_limit_bytes=62 * 1024 * 1024),
    )(partial, resid, gamma2)
